# baseline (speedup 1.0000x reference)
import jax
import jax.numpy as jnp
from jax.experimental import pallas as pl


_H = 256
_W = 256
_KC = 19


def _fused_kernel(hm_ref, paf_ref, out_ref):
    t = hm_ref[0]

    kc = t.shape[0]
    ninf = jnp.full((kc, 1, _W), -jnp.inf, dtype=t.dtype)
    ninfc = jnp.full((kc, _H, 1), -jnp.inf, dtype=t.dtype)
    nxt_col = jnp.concatenate([t[:, :, 1:], ninfc], axis=2)
    prv_col = jnp.concatenate([ninfc, t[:, :, :-1]], axis=2)
    nxt_row = jnp.concatenate([t[:, 1:, :], ninf], axis=1)
    prv_row = jnp.concatenate([ninf, t[:, :-1, :]], axis=1)

    nmax = jnp.maximum(jnp.maximum(nxt_col, prv_col),
                       jnp.maximum(nxt_row, prv_row))
    peak = (t > nmax) & (t >= 0.1)
    out_ref[0, 0] = jnp.where(peak, t, 0.0)

    px = paf_ref[0, :, 0]
    py = paf_ref[0, :, 1]
    out_ref[0, 1] = jnp.sqrt(px * px + py * py + 1e-12)


def kernel(heatmap2d, paf2d):
    B, K, H, W = heatmap2d.shape
    paf = paf2d.reshape(B, K, 2, H, W)

    out = pl.pallas_call(
        _fused_kernel,
        grid=(B, K // _KC),
        in_specs=[
            pl.BlockSpec((1, _KC, H, W), lambda b, k: (b, k, 0, 0)),
            pl.BlockSpec((1, _KC, 2, H, W), lambda b, k: (b, k, 0, 0, 0)),
        ],
        out_specs=pl.BlockSpec((1, 2, _KC, H, W), lambda b, k: (b, 0, k, 0, 0)),
        out_shape=jax.ShapeDtypeStruct((B, 2, K, H, W), heatmap2d.dtype),
    )(heatmap2d, paf)

    return out.reshape(B, 2 * K, H, W)
